# Initial kernel scaffold; baseline (speedup 1.0000x reference)
import jax
import jax.numpy as jnp
from jax.experimental import pallas as pl


def kernel(hand_xyz, hand_normal, obj_xyz):
    raise NotImplementedError("write your pallas kernel here")


# speedup vs baseline: 14.6271x; 14.6271x over previous
import functools

import jax
import jax.numpy as jnp
from jax import lax
from jax.experimental import pallas as pl
from jax.experimental.pallas import tpu as pltpu
from jax.experimental.pallas import tpu_sc as plsc

_MBLK = 376


def _tc_knn_body(o_ref, h_ref, nnd_ref, nni_ref):
    o = o_ref[0]
    h = h_ref[0]
    g = jnp.dot(o, h, preferred_element_type=jnp.float32)
    h2 = jnp.sum(h * h, axis=0, keepdims=True)
    t = h2 - 2.0 * g
    o2 = jnp.sum(o * o, axis=1, keepdims=True)
    mn = jnp.min(t, axis=1, keepdims=True)
    ji = lax.broadcasted_iota(jnp.int32, t.shape, 1)
    idx = jnp.min(jnp.where(t == mn, ji, jnp.int32(2**30)), axis=1)
    nnd_ref[0, 0, :] = mn[:, 0] + o2[:, 0]
    nni_ref[0, 0, :] = idx


def _sc_interior_body(No, hand6_h, obj3_h, nnd_h, nni_h, out_h,
                      hx_v, hy_v, hz_v, nx_v, ny_v, nz_v,
                      ox_v, oy_v, oz_v, nnd_v, nni_v, acc_v):
    w = lax.axis_index("s") * 2 + lax.axis_index("c")
    NhG = hx_v.shape[0]
    NoP = nnd_v.shape[0]
    for r, ref in enumerate((hx_v, hy_v, hz_v, nx_v, ny_v, nz_v)):
        pltpu.sync_copy(hand6_h.at[pl.ds((w * 6 + r) * NhG, NhG)], ref)
    for r, ref in enumerate((ox_v, oy_v, oz_v)):
        pltpu.sync_copy(obj3_h.at[pl.ds((w * 3 + r) * NoP, NoP)], ref)
    pltpu.sync_copy(nnd_h.at[pl.ds(w * NoP, NoP)], nnd_v)
    pltpu.sync_copy(nni_h.at[pl.ds(w * NoP, NoP)], nni_v)
    lane = lax.iota(jnp.int32, 16)

    def body(i, acc):
        st = i * 16
        idx = nni_v[pl.ds(st, 16)]
        gx = plsc.load_gather(hx_v, [idx])
        gy = plsc.load_gather(hy_v, [idx])
        gz = plsc.load_gather(hz_v, [idx])
        nx = plsc.load_gather(nx_v, [idx])
        ny = plsc.load_gather(ny_v, [idx])
        nz = plsc.load_gather(nz_v, [idx])
        ox = ox_v[pl.ds(st, 16)]
        oy = oy_v[pl.ds(st, 16)]
        oz = oz_v[pl.ds(st, 16)]
        score = (gx - ox) * nx + (gy - oy) * ny + (gz - oz) * nz
        nnd = nnd_v[pl.ds(st, 16)]
        keep = jnp.logical_and(score > 0.0, (st + lane) < No)
        return acc + jnp.where(keep, nnd, 0.0)

    acc = lax.fori_loop(0, nnd_v.shape[0] // 16, body,
                        jnp.zeros((16,), jnp.float32))
    acc_v[...] = acc
    pltpu.sync_copy(acc_v, out_h.at[pl.ds(w * 16, 16)])


def kernel(hand_xyz, hand_normal, obj_xyz):
    B, Nh, _ = hand_xyz.shape
    No = obj_xyz.shape[1]
    NhP = 896
    NhG = 784
    NoP = 3008
    nblk = NoP // _MBLK

    f32 = jnp.float32
    hand_t = jnp.transpose(hand_xyz, (0, 2, 1))
    norm_t = jnp.transpose(hand_normal, (0, 2, 1))
    obj_t = jnp.transpose(obj_xyz, (0, 2, 1))

    hand_p = jnp.zeros((B, 8, NhP), f32)
    hand_p = hand_p.at[:, :3, :Nh].set(hand_t)
    hand_p = hand_p.at[:, 0, Nh:].set(1e9)
    obj_p = jnp.zeros((B, NoP, 8), f32)
    obj_p = obj_p.at[:, :No, :3].set(obj_xyz)

    nnd, nni = pl.pallas_call(
        _tc_knn_body,
        grid=(B, nblk),
        in_specs=[
            pl.BlockSpec((1, _MBLK, 8), lambda b, m: (b, m, 0)),
            pl.BlockSpec((1, 8, NhP), lambda b, m: (b, 0, 0)),
        ],
        out_specs=[
            pl.BlockSpec((1, 1, _MBLK), lambda b, m: (b * nblk + m, 0, 0)),
            pl.BlockSpec((1, 1, _MBLK), lambda b, m: (b * nblk + m, 0, 0)),
        ],
        out_shape=[
            jax.ShapeDtypeStruct((B * nblk, 1, _MBLK), f32),
            jax.ShapeDtypeStruct((B * nblk, 1, _MBLK), jnp.int32),
        ],
    )(obj_p, hand_p)
    nnd = nnd.reshape(B, NoP)
    nni = nni.reshape(B, NoP)

    hand6 = jnp.zeros((B, 6, NhG), f32)
    hand6 = hand6.at[:, :3, :Nh].set(hand_t)
    hand6 = hand6.at[:, 3:, :Nh].set(norm_t)
    obj3 = jnp.zeros((B, 3, NoP), f32)
    obj3 = obj3.at[:, :, :No].set(obj_t)

    mesh = plsc.VectorSubcoreMesh(core_axis_name="c", subcore_axis_name="s")
    sc = functools.partial(
        pl.kernel,
        mesh=mesh,
        compiler_params=pltpu.CompilerParams(needs_layout_passes=False),
        out_type=jax.ShapeDtypeStruct((B * 16,), f32),
        scratch_types=(
            [pltpu.VMEM((NhG,), f32)] * 6
            + [pltpu.VMEM((NoP,), f32)] * 3
            + [pltpu.VMEM((NoP,), f32), pltpu.VMEM((NoP,), jnp.int32),
               pltpu.VMEM((16,), f32)]
        ),
    )(functools.partial(_sc_interior_body, No))
    partials = sc(hand6.reshape(-1), obj3.reshape(-1),
                  nnd.reshape(-1), nni.reshape(-1))

    return jnp.sum(partials) / B
